# Initial kernel scaffold; baseline (speedup 1.0000x reference)
import jax
import jax.numpy as jnp
from jax.experimental import pallas as pl


def kernel(x, hyperedge_index, hyperedge_attr, W, att, bias, gamma, beta):
    raise NotImplementedError("write your pallas kernel here")


# speedup vs baseline: 14.2474x; 14.2474x over previous
import functools

import jax
import jax.numpy as jnp
from jax import lax
from jax.experimental import pallas as pl
from jax.experimental.pallas import tpu as pltpu
from jax.experimental.pallas import tpu_sc as plsc

N = 10000
E = 10000
INC = 320000
D = 128
NC, NS = 2, 16
NW = NC * NS
PW = INC // NW
K = 200
NCH = PW // K
RB = 2000
G = N // RB
EP = 10112
SP = EP // NS
KP = 256
DUMMY = N
DH = 64

_mesh = plsc.VectorSubcoreMesh(
    core_axis_name="c", subcore_axis_name="s", num_cores=NC, num_subcores=NS)
_sc_params = pltpu.CompilerParams(needs_layout_passes=False,
                                  use_tc_tiling_on_sc=False)


def _proj_body(x_ref, ha_ref, w_ref, att_ref, xl_ref, ea_ref, s1_ref, s2_ref):
    w = w_ref[...]
    xl = lax.dot_general(x_ref[...], w, (((1,), (1,)), ((), ())),
                         preferred_element_type=jnp.float32)
    ea = lax.dot_general(ha_ref[...], w, (((1,), (1,)), ((), ())),
                         preferred_element_type=jnp.float32)
    xl_ref[...] = xl
    ea_ref[...] = ea
    s1_ref[0, 0, :] = jnp.sum(xl * att_ref[0, :][None, :], axis=1)
    s2_ref[0, 0, :] = jnp.sum(ea * att_ref[1, :][None, :], axis=1)


def _project(x, ha, w, att2d):
    return pl.pallas_call(
        _proj_body,
        grid=(G,),
        in_specs=[
            pl.BlockSpec((RB, D), lambda i: (i, 0)),
            pl.BlockSpec((RB, D), lambda i: (i, 0)),
            pl.BlockSpec((D, D), lambda i: (0, 0)),
            pl.BlockSpec((2, D), lambda i: (0, 0)),
        ],
        out_specs=[
            pl.BlockSpec((RB, D), lambda i: (i, 0)),
            pl.BlockSpec((RB, D), lambda i: (i, 0)),
            pl.BlockSpec((1, 1, RB), lambda i: (i, 0, 0)),
            pl.BlockSpec((1, 1, RB), lambda i: (i, 0, 0)),
        ],
        out_shape=[
            jax.ShapeDtypeStruct((N, D), jnp.float32),
            jax.ShapeDtypeStruct((E, D), jnp.float32),
            jax.ShapeDtypeStruct((G, 1, RB), jnp.float32),
            jax.ShapeDtypeStruct((G, 1, RB), jnp.float32),
        ],
    )(x, ha, w, att2d)


@functools.partial(
    pl.kernel,
    out_type=[
        jax.ShapeDtypeStruct((INC,), jnp.float32),
        jax.ShapeDtypeStruct((NW * N,), jnp.float32),
        jax.ShapeDtypeStruct((NW * E,), jnp.float32),
        jax.ShapeDtypeStruct((NW * E,), jnp.float32),
    ],
    mesh=_mesh,
    compiler_params=_sc_params,
    scratch_types=[
        pltpu.VMEM((PW,), jnp.int32),
        pltpu.VMEM((PW,), jnp.int32),
        pltpu.VMEM((N,), jnp.float32),
        pltpu.VMEM((E,), jnp.float32),
        pltpu.VMEM((N,), jnp.float32),
        pltpu.VMEM((E,), jnp.float32),
        pltpu.VMEM((E,), jnp.float32),
        pltpu.VMEM((PW,), jnp.float32),
    ],
)
def _sc_scores(row_hbm, col_hbm, s1_hbm, s2_hbm,
               ex_hbm, dpart_hbm, bpart_hbm, epart_hbm,
               row_v, col_v, s1_v, s2_v, d_v, b_v, e_v, ex_v):
    wid = lax.axis_index("s") * NC + lax.axis_index("c")
    base = pl.multiple_of(wid * PW, 8)
    pltpu.sync_copy(row_hbm.at[pl.ds(base, PW)], row_v)
    pltpu.sync_copy(col_hbm.at[pl.ds(base, PW)], col_v)
    pltpu.sync_copy(s1_hbm, s1_v)
    pltpu.sync_copy(s2_hbm, s2_v)

    z = jnp.zeros((16,), jnp.float32)

    def zbody(i, carry):
        d_v[pl.ds(i * 16, 16)] = z
        b_v[pl.ds(i * 16, 16)] = z
        e_v[pl.ds(i * 16, 16)] = z
        return carry

    lax.fori_loop(0, N // 16, zbody, 0)

    one = jnp.ones((16,), jnp.float32)

    def body(g, carry):
        r = row_v[pl.ds(g * 16, 16)]
        c = col_v[pl.ds(g * 16, 16)]
        a = plsc.load_gather(s1_v, [r]) + plsc.load_gather(s2_v, [c])
        a = jnp.where(a > 0, a, 0.2 * a)
        ev = jnp.exp(a)
        ex_v[pl.ds(g * 16, 16)] = ev
        plsc.addupdate_scatter(e_v, [c], ev)
        plsc.addupdate_scatter(d_v, [r], one)
        plsc.addupdate_scatter(b_v, [c], one)
        return carry

    lax.fori_loop(0, PW // 16, body, 0)

    pltpu.sync_copy(ex_v, ex_hbm.at[pl.ds(base, PW)])
    pltpu.sync_copy(d_v, dpart_hbm.at[pl.ds(pl.multiple_of(wid * N, 8), N)])
    pltpu.sync_copy(b_v, bpart_hbm.at[pl.ds(pl.multiple_of(wid * E, 8), E)])
    pltpu.sync_copy(e_v, epart_hbm.at[pl.ds(pl.multiple_of(wid * E, 8), E)])


def _norm_body(dp_ref, bp_ref, ep_ref, dinv_ref, be_ref, einv_ref):
    dsum = jnp.sum(dp_ref[...], axis=0)
    bsum = jnp.sum(bp_ref[...], axis=0)
    esum = jnp.sum(ep_ref[...], axis=0)
    dinv = jnp.where(dsum > 0, 1.0 / dsum, 0.0)
    binv = jnp.where(bsum > 0, 1.0 / bsum, 0.0)
    einv = 1.0 / (esum + 1e-16)
    dinv_ref[0, 0, :] = dinv
    be_ref[0, 0, :] = binv * einv
    einv_ref[0, 0, :] = einv


def _norm_tables(dpart, bpart, epart):
    return pl.pallas_call(
        _norm_body,
        out_shape=[
            jax.ShapeDtypeStruct((1, 1, N), jnp.float32),
            jax.ShapeDtypeStruct((1, 1, E), jnp.float32),
            jax.ShapeDtypeStruct((1, 1, E), jnp.float32),
        ],
    )(dpart, bpart, epart)


@functools.partial(
    pl.kernel,
    out_type=[
        jax.ShapeDtypeStruct((INC,), jnp.float32),
        jax.ShapeDtypeStruct((INC,), jnp.float32),
    ],
    mesh=_mesh,
    compiler_params=_sc_params,
    scratch_types=[
        pltpu.VMEM((PW,), jnp.int32),
        pltpu.VMEM((PW,), jnp.int32),
        pltpu.VMEM((PW,), jnp.float32),
        pltpu.VMEM((E,), jnp.float32),
        pltpu.VMEM((E,), jnp.float32),
        pltpu.VMEM((N,), jnp.float32),
        pltpu.VMEM((PW,), jnp.float32),
        pltpu.VMEM((PW,), jnp.float32),
    ],
)
def _sc_weights(row_hbm, col_hbm, ex_hbm, be_hbm, einv_hbm, dinv_hbm,
                w1_hbm, w2_hbm,
                row_v, col_v, ex_v, be_v, einv_v, dinv_v, w1_v, w2_v):
    wid = lax.axis_index("s") * NC + lax.axis_index("c")
    base = pl.multiple_of(wid * PW, 8)
    pltpu.sync_copy(row_hbm.at[pl.ds(base, PW)], row_v)
    pltpu.sync_copy(col_hbm.at[pl.ds(base, PW)], col_v)
    pltpu.sync_copy(ex_hbm.at[pl.ds(base, PW)], ex_v)
    pltpu.sync_copy(be_hbm, be_v)
    pltpu.sync_copy(einv_hbm, einv_v)
    pltpu.sync_copy(dinv_hbm, dinv_v)

    def wbody(g, carry):
        r = row_v[pl.ds(g * 16, 16)]
        c = col_v[pl.ds(g * 16, 16)]
        e = ex_v[pl.ds(g * 16, 16)]
        ei = plsc.load_gather(einv_v, [c])
        w1_v[pl.ds(g * 16, 16)] = plsc.load_gather(be_v, [c]) * e
        w2_v[pl.ds(g * 16, 16)] = plsc.load_gather(dinv_v, [r]) * (ei * e)
        return carry

    lax.fori_loop(0, PW // 16, wbody, 0)
    pltpu.sync_copy(w1_v, w1_hbm.at[pl.ds(base, PW)])
    pltpu.sync_copy(w2_v, w2_hbm.at[pl.ds(base, PW)])


@functools.partial(
    pl.kernel,
    out_type=jax.ShapeDtypeStruct((NC * EP, DH), jnp.float32),
    mesh=_mesh,
    compiler_params=_sc_params,
    scratch_types=[
        pltpu.VMEM((PW,), jnp.int32),
        pltpu.VMEM((KP,), jnp.int32),
        pltpu.VMEM((PW,), jnp.float32),
        pltpu.VMEM((KP, DH), jnp.float32),
        pltpu.VMEM_SHARED((EP, DH), jnp.float32),
    ],
)
def _sc_spmv(gidx_hbm, sidx_hbm, w_hbm, table_hbm, z_hbm,
             part_hbm,
             gidx_v, cidx_v, w_v, buf, acc):
    cid = lax.axis_index("c")
    sid = lax.axis_index("s")
    wid = sid * NC + cid
    base = pl.multiple_of(wid * PW, 8)
    pltpu.sync_copy(gidx_hbm.at[pl.ds(base, PW)], gidx_v)
    pltpu.sync_copy(w_hbm.at[pl.ds(base, PW)], w_v)

    stripe = pl.multiple_of(sid * SP, 8)
    pltpu.sync_copy(z_hbm.at[pl.ds(stripe, SP)], acc.at[pl.ds(stripe, SP)])

    zv = jnp.zeros((16,), jnp.float32)

    def zpad(k, carry):
        for j in range(DH // 16):
            buf[K + k, pl.ds(j * 16, 16)] = zv
        return carry

    lax.fori_loop(0, KP - K, zpad, 0)
    plsc.subcore_barrier()

    def chunk(ch, carry):
        off = pl.multiple_of(ch * K, 8)
        off3 = pl.multiple_of((wid * NCH + ch) * KP, 8)
        pltpu.sync_copy(sidx_hbm.at[pl.ds(off3, KP)], cidx_v)
        pltpu.sync_copy(table_hbm.at[gidx_v.at[pl.ds(off, K)]],
                        buf.at[pl.ds(0, K)])

        def scale(k, c2):
            wv = plsc.load_gather(w_v, [jnp.full((16,), off + k, jnp.int32)])
            for j in range(DH // 16):
                buf[k, pl.ds(j * 16, 16)] = buf[k, pl.ds(j * 16, 16)] * wv
            return c2

        lax.fori_loop(0, K, scale, 0)
        pltpu.sync_copy(buf, acc.at[cidx_v], add=True)
        return carry

    lax.fori_loop(0, NCH, chunk, 0)
    plsc.subcore_barrier()
    pltpu.sync_copy(
        acc.at[pl.ds(stripe, SP)],
        part_hbm.at[pl.ds(pl.multiple_of(cid * EP + sid * SP, 8), SP)])


def _comb_body(p0_ref, p1_ref, o_ref):
    o_ref[...] = p0_ref[0] + p1_ref[0]


def _combine(parts):
    return pl.pallas_call(
        _comb_body,
        grid=(G,),
        in_specs=[
            pl.BlockSpec((1, RB, DH), lambda i: (0, i, 0)),
            pl.BlockSpec((1, RB, DH), lambda i: (1, i, 0)),
        ],
        out_specs=pl.BlockSpec((RB, DH), lambda i: (i, 0)),
        out_shape=jax.ShapeDtypeStruct((E, DH), jnp.float32),
    )(parts, parts)


def _bn_body(p0l_ref, p1l_ref, p0h_ref, p1h_ref, x_ref, b_ref, g_ref,
             be_ref, o_ref, s_ref, ss_ref):
    p = pl.program_id(0)
    i = pl.program_id(1)
    h = jnp.concatenate(
        [p0l_ref[0] + p1l_ref[0], p0h_ref[0] + p1h_ref[0]], axis=1)
    h = h + b_ref[...]

    @pl.when(jnp.logical_and(p == 0, i == 0))
    def _init():
        s_ref[...] = jnp.zeros_like(s_ref)
        ss_ref[...] = jnp.zeros_like(ss_ref)

    @pl.when(p == 0)
    def _accum():
        s_ref[0:1, :] += jnp.sum(h, axis=0, keepdims=True)
        ss_ref[0:1, :] += jnp.sum(h * h, axis=0, keepdims=True)

    @pl.when(p == 1)
    def _apply():
        mean = s_ref[0:1, :] / N
        var = ss_ref[0:1, :] / N - mean * mean
        t = (h - mean) * (g_ref[...] / jnp.sqrt(var + 1e-5)) + be_ref[...]
        t = jnp.where(t > 0, t, jnp.exp(t) - 1.0)
        o_ref[...] = t + x_ref[...]


def _bn_elu_res(parts_lo, parts_hi, x, bias2d, gamma2d, beta2d):
    return pl.pallas_call(
        _bn_body,
        grid=(2, G),
        in_specs=[
            pl.BlockSpec((1, RB, DH), lambda p, i: (0, i, 0)),
            pl.BlockSpec((1, RB, DH), lambda p, i: (1, i, 0)),
            pl.BlockSpec((1, RB, DH), lambda p, i: (0, i, 0)),
            pl.BlockSpec((1, RB, DH), lambda p, i: (1, i, 0)),
            pl.BlockSpec((RB, D), lambda p, i: (i, 0)),
            pl.BlockSpec((1, D), lambda p, i: (0, 0)),
            pl.BlockSpec((1, D), lambda p, i: (0, 0)),
            pl.BlockSpec((1, D), lambda p, i: (0, 0)),
        ],
        out_specs=pl.BlockSpec((RB, D), lambda p, i: (i, 0)),
        out_shape=jax.ShapeDtypeStruct((N, D), jnp.float32),
        scratch_shapes=[
            pltpu.VMEM((8, D), jnp.float32),
            pltpu.VMEM((8, D), jnp.float32),
        ],
    )(parts_lo, parts_lo, parts_hi, parts_hi, x, bias2d, gamma2d, beta2d)


def kernel(x, hyperedge_index, hyperedge_attr, W, att, bias, gamma, beta):
    row = hyperedge_index[0].astype(jnp.int32)
    col = hyperedge_index[1].astype(jnp.int32)
    att2d = att.reshape(2, D)

    xl, ea, s1b, s2b = _project(x, hyperedge_attr, W, att2d)
    s1 = s1b.reshape(N)
    s2 = s2b.reshape(E)

    ex, dpart, bpart, epart = _sc_scores(row, col, s1, s2)
    dinv3, be3, einv3 = _norm_tables(dpart.reshape(NW, N), bpart.reshape(NW, E),
                                     epart.reshape(NW, E))
    dinv = dinv3.reshape(N)
    be = be3.reshape(E)
    einv = einv3.reshape(E)

    w1, w2 = _sc_weights(row, col, ex, be, einv, dinv)

    pad = jnp.full((NW, NCH, KP - K), DUMMY, jnp.int32)
    col3 = jnp.concatenate([col.reshape(NW, NCH, K), pad], axis=2).reshape(-1)
    row3 = jnp.concatenate([row.reshape(NW, NCH, K), pad], axis=2).reshape(-1)
    zeros = jnp.zeros((EP, DH), jnp.float32)

    outn_halves = []
    for h in range(D // DH):
        xl_h = lax.slice(xl, (0, h * DH), (N, (h + 1) * DH))
        oute2 = _sc_spmv(row, col3, w1, xl_h, zeros)
        oute_h = _combine(oute2.reshape(NC, EP, DH))
        outn_halves.append(_sc_spmv(col, row3, w2, oute_h, zeros))

    out = _bn_elu_res(outn_halves[0].reshape(NC, EP, DH),
                      outn_halves[1].reshape(NC, EP, DH), x,
                      bias.reshape(1, D), gamma.reshape(1, D),
                      beta.reshape(1, D))
    return out


# speedup vs baseline: 22.1710x; 1.5561x over previous
import functools

import jax
import jax.numpy as jnp
from jax import lax
from jax.experimental import pallas as pl
from jax.experimental.pallas import tpu as pltpu
from jax.experimental.pallas import tpu_sc as plsc

N = 10000
E = 10000
INC = 320000
D = 128
NC, NS = 2, 16
NW = NC * NS
PW = INC // NW
K = 200
NCH = PW // K
RB = 2000
G = N // RB
EP = 10112
SP = EP // NS
KP = 256
DUMMY = N
DH = 64

_mesh = plsc.VectorSubcoreMesh(
    core_axis_name="c", subcore_axis_name="s", num_cores=NC, num_subcores=NS)
_sc_params = pltpu.CompilerParams(needs_layout_passes=False,
                                  use_tc_tiling_on_sc=False)


def _proj_body(x_ref, ha_ref, w_ref, att_ref, xl_ref, ea_ref, s1_ref, s2_ref):
    w = w_ref[...]
    xl = lax.dot_general(x_ref[...], w, (((1,), (1,)), ((), ())),
                         preferred_element_type=jnp.float32)
    ea = lax.dot_general(ha_ref[...], w, (((1,), (1,)), ((), ())),
                         preferred_element_type=jnp.float32)
    xl_ref[...] = xl
    ea_ref[...] = ea
    s1_ref[0, 0, :] = jnp.sum(xl * att_ref[0, :][None, :], axis=1)
    s2_ref[0, 0, :] = jnp.sum(ea * att_ref[1, :][None, :], axis=1)


def _project(x, ha, w, att2d):
    return pl.pallas_call(
        _proj_body,
        grid=(G,),
        in_specs=[
            pl.BlockSpec((RB, D), lambda i: (i, 0)),
            pl.BlockSpec((RB, D), lambda i: (i, 0)),
            pl.BlockSpec((D, D), lambda i: (0, 0)),
            pl.BlockSpec((2, D), lambda i: (0, 0)),
        ],
        out_specs=[
            pl.BlockSpec((RB, D), lambda i: (i, 0)),
            pl.BlockSpec((RB, D), lambda i: (i, 0)),
            pl.BlockSpec((1, 1, RB), lambda i: (i, 0, 0)),
            pl.BlockSpec((1, 1, RB), lambda i: (i, 0, 0)),
        ],
        out_shape=[
            jax.ShapeDtypeStruct((N, D), jnp.float32),
            jax.ShapeDtypeStruct((E, D), jnp.float32),
            jax.ShapeDtypeStruct((G, 1, RB), jnp.float32),
            jax.ShapeDtypeStruct((G, 1, RB), jnp.float32),
        ],
    )(x, ha, w, att2d)


@functools.partial(
    pl.kernel,
    out_type=[
        jax.ShapeDtypeStruct((INC,), jnp.float32),
        jax.ShapeDtypeStruct((NW * N,), jnp.float32),
        jax.ShapeDtypeStruct((NW * E,), jnp.float32),
        jax.ShapeDtypeStruct((NW * E,), jnp.float32),
    ],
    mesh=_mesh,
    compiler_params=_sc_params,
    scratch_types=[
        pltpu.VMEM((PW,), jnp.int32),
        pltpu.VMEM((PW,), jnp.int32),
        pltpu.VMEM((N,), jnp.float32),
        pltpu.VMEM((E,), jnp.float32),
        pltpu.VMEM((N,), jnp.float32),
        pltpu.VMEM((E,), jnp.float32),
        pltpu.VMEM((E,), jnp.float32),
        pltpu.VMEM((PW,), jnp.float32),
    ],
)
def _sc_scores(row_hbm, col_hbm, s1_hbm, s2_hbm,
               ex_hbm, dpart_hbm, bpart_hbm, epart_hbm,
               row_v, col_v, s1_v, s2_v, d_v, b_v, e_v, ex_v):
    wid = lax.axis_index("s") * NC + lax.axis_index("c")
    base = pl.multiple_of(wid * PW, 8)
    pltpu.sync_copy(row_hbm.at[pl.ds(base, PW)], row_v)
    pltpu.sync_copy(col_hbm.at[pl.ds(base, PW)], col_v)
    pltpu.sync_copy(s1_hbm, s1_v)
    pltpu.sync_copy(s2_hbm, s2_v)

    z = jnp.zeros((16,), jnp.float32)

    def zbody(i, carry):
        d_v[pl.ds(i * 16, 16)] = z
        b_v[pl.ds(i * 16, 16)] = z
        e_v[pl.ds(i * 16, 16)] = z
        return carry

    lax.fori_loop(0, N // 16, zbody, 0)

    one = jnp.ones((16,), jnp.float32)

    def body(g, carry):
        r = row_v[pl.ds(g * 16, 16)]
        c = col_v[pl.ds(g * 16, 16)]
        a = plsc.load_gather(s1_v, [r]) + plsc.load_gather(s2_v, [c])
        a = jnp.where(a > 0, a, 0.2 * a)
        ev = jnp.exp(a)
        ex_v[pl.ds(g * 16, 16)] = ev
        plsc.addupdate_scatter(e_v, [c], ev)
        plsc.addupdate_scatter(d_v, [r], one)
        plsc.addupdate_scatter(b_v, [c], one)
        return carry

    lax.fori_loop(0, PW // 16, body, 0)

    pltpu.sync_copy(ex_v, ex_hbm.at[pl.ds(base, PW)])
    pltpu.sync_copy(d_v, dpart_hbm.at[pl.ds(pl.multiple_of(wid * N, 8), N)])
    pltpu.sync_copy(b_v, bpart_hbm.at[pl.ds(pl.multiple_of(wid * E, 8), E)])
    pltpu.sync_copy(e_v, epart_hbm.at[pl.ds(pl.multiple_of(wid * E, 8), E)])


def _norm_body(dp_ref, bp_ref, ep_ref, dinv_ref, be_ref, einv_ref):
    dsum = jnp.sum(dp_ref[...], axis=0)
    bsum = jnp.sum(bp_ref[...], axis=0)
    esum = jnp.sum(ep_ref[...], axis=0)
    dinv = jnp.where(dsum > 0, 1.0 / dsum, 0.0)
    binv = jnp.where(bsum > 0, 1.0 / bsum, 0.0)
    einv = 1.0 / (esum + 1e-16)
    dinv_ref[0, 0, :] = dinv
    be_ref[0, 0, :] = binv * einv
    einv_ref[0, 0, :] = einv


def _norm_tables(dpart, bpart, epart):
    return pl.pallas_call(
        _norm_body,
        out_shape=[
            jax.ShapeDtypeStruct((1, 1, N), jnp.float32),
            jax.ShapeDtypeStruct((1, 1, E), jnp.float32),
            jax.ShapeDtypeStruct((1, 1, E), jnp.float32),
        ],
    )(dpart, bpart, epart)


@functools.partial(
    pl.kernel,
    out_type=[
        jax.ShapeDtypeStruct((INC,), jnp.float32),
        jax.ShapeDtypeStruct((INC,), jnp.float32),
    ],
    mesh=_mesh,
    compiler_params=_sc_params,
    scratch_types=[
        pltpu.VMEM((PW,), jnp.int32),
        pltpu.VMEM((PW,), jnp.int32),
        pltpu.VMEM((PW,), jnp.float32),
        pltpu.VMEM((E,), jnp.float32),
        pltpu.VMEM((E,), jnp.float32),
        pltpu.VMEM((N,), jnp.float32),
        pltpu.VMEM((PW,), jnp.float32),
        pltpu.VMEM((PW,), jnp.float32),
    ],
)
def _sc_weights(row_hbm, col_hbm, ex_hbm, be_hbm, einv_hbm, dinv_hbm,
                w1_hbm, w2_hbm,
                row_v, col_v, ex_v, be_v, einv_v, dinv_v, w1_v, w2_v):
    wid = lax.axis_index("s") * NC + lax.axis_index("c")
    base = pl.multiple_of(wid * PW, 8)
    pltpu.sync_copy(row_hbm.at[pl.ds(base, PW)], row_v)
    pltpu.sync_copy(col_hbm.at[pl.ds(base, PW)], col_v)
    pltpu.sync_copy(ex_hbm.at[pl.ds(base, PW)], ex_v)
    pltpu.sync_copy(be_hbm, be_v)
    pltpu.sync_copy(einv_hbm, einv_v)
    pltpu.sync_copy(dinv_hbm, dinv_v)

    def wbody(g, carry):
        r = row_v[pl.ds(g * 16, 16)]
        c = col_v[pl.ds(g * 16, 16)]
        e = ex_v[pl.ds(g * 16, 16)]
        ei = plsc.load_gather(einv_v, [c])
        w1_v[pl.ds(g * 16, 16)] = plsc.load_gather(be_v, [c]) * e
        w2_v[pl.ds(g * 16, 16)] = plsc.load_gather(dinv_v, [r]) * (ei * e)
        return carry

    lax.fori_loop(0, PW // 16, wbody, 0)
    pltpu.sync_copy(w1_v, w1_hbm.at[pl.ds(base, PW)])
    pltpu.sync_copy(w2_v, w2_hbm.at[pl.ds(base, PW)])


@functools.partial(
    pl.kernel,
    out_type=jax.ShapeDtypeStruct((NC * EP, DH), jnp.float32),
    mesh=_mesh,
    compiler_params=_sc_params,
    scratch_types=[
        pltpu.VMEM((PW,), jnp.int32),
        pltpu.VMEM((PW,), jnp.float32),
        pltpu.VMEM((KP, DH), jnp.float32),
        pltpu.VMEM((KP, DH), jnp.float32),
        pltpu.VMEM((KP,), jnp.int32),
        pltpu.VMEM((KP,), jnp.int32),
        pltpu.SemaphoreType.DMA,
        pltpu.SemaphoreType.DMA,
        pltpu.SemaphoreType.DMA,
        pltpu.SemaphoreType.DMA,
        pltpu.VMEM_SHARED((EP, DH), jnp.float32),
    ],
)
def _sc_spmv(gidx_hbm, sidx_hbm, w_hbm, table_hbm, z_hbm,
             part_hbm,
             gidx_v, w_v, buf_a, buf_b, cidx_a, cidx_b,
             gsem_a, gsem_b, csem_a, csem_b, acc):
    cid = lax.axis_index("c")
    sid = lax.axis_index("s")
    wid = sid * NC + cid
    base = pl.multiple_of(wid * PW, 8)
    pltpu.sync_copy(gidx_hbm.at[pl.ds(base, PW)], gidx_v)
    pltpu.sync_copy(w_hbm.at[pl.ds(base, PW)], w_v)

    stripe = pl.multiple_of(sid * SP, 8)
    pltpu.sync_copy(z_hbm.at[pl.ds(stripe, SP)], acc.at[pl.ds(stripe, SP)])

    zv = jnp.zeros((16,), jnp.float32)

    def zpad(k, carry):
        for j in range(DH // 16):
            buf_a[K + k, pl.ds(j * 16, 16)] = zv
            buf_b[K + k, pl.ds(j * 16, 16)] = zv
        return carry

    lax.fori_loop(0, KP - K, zpad, 0)
    plsc.subcore_barrier()

    slots = ((buf_a, cidx_a, gsem_a, csem_a), (buf_b, cidx_b, gsem_b, csem_b))

    def issue(cc, buf, cidx, gsem, csem):
        off = pl.multiple_of(cc * K, 8)
        off3 = pl.multiple_of((wid * NCH + cc) * KP, 8)
        pltpu.async_copy(sidx_hbm.at[pl.ds(off3, KP)], cidx, csem)
        pltpu.async_copy(table_hbm.at[gidx_v.at[pl.ds(off, K)]],
                         buf.at[pl.ds(0, K)], gsem)

    for b in range(2):
        buf, cidx, gsem, csem = slots[b]
        issue(b, buf, cidx, gsem, csem)

    def pair(p, carry):
        ch = p * 2
        for b in range(2):
            cc = ch + b
            buf, cidx, gsem, csem = slots[b]
            off = pl.multiple_of(cc * K, 8)
            pltpu.make_async_copy(sidx_hbm.at[pl.ds(0, KP)], cidx, csem).wait()
            pltpu.make_async_copy(table_hbm.at[gidx_v.at[pl.ds(0, K)]],
                                  buf.at[pl.ds(0, K)], gsem).wait()

            def scale(k, c2):
                wv = plsc.load_gather(
                    w_v, [jnp.full((16,), off + k, jnp.int32)])
                for j in range(DH // 16):
                    buf[k, pl.ds(j * 16, 16)] = buf[k, pl.ds(j * 16, 16)] * wv
                return c2

            lax.fori_loop(0, K, scale, 0, unroll=8)
            pltpu.sync_copy(buf, acc.at[cidx], add=True)

            @pl.when(cc + 2 < NCH)
            def _():
                issue(cc + 2, buf, cidx, gsem, csem)
        return carry

    lax.fori_loop(0, NCH // 2, pair, 0)
    plsc.subcore_barrier()
    pltpu.sync_copy(
        acc.at[pl.ds(stripe, SP)],
        part_hbm.at[pl.ds(pl.multiple_of(cid * EP + sid * SP, 8), SP)])


def _comb_body(p0_ref, p1_ref, o_ref):
    o_ref[...] = p0_ref[0] + p1_ref[0]


def _combine(parts):
    return pl.pallas_call(
        _comb_body,
        grid=(G,),
        in_specs=[
            pl.BlockSpec((1, RB, DH), lambda i: (0, i, 0)),
            pl.BlockSpec((1, RB, DH), lambda i: (1, i, 0)),
        ],
        out_specs=pl.BlockSpec((RB, DH), lambda i: (i, 0)),
        out_shape=jax.ShapeDtypeStruct((E, DH), jnp.float32),
    )(parts, parts)


def _bn_body(p0l_ref, p1l_ref, p0h_ref, p1h_ref, x_ref, b_ref, g_ref,
             be_ref, o_ref, s_ref, ss_ref):
    p = pl.program_id(0)
    i = pl.program_id(1)
    h = jnp.concatenate(
        [p0l_ref[0] + p1l_ref[0], p0h_ref[0] + p1h_ref[0]], axis=1)
    h = h + b_ref[...]

    @pl.when(jnp.logical_and(p == 0, i == 0))
    def _init():
        s_ref[...] = jnp.zeros_like(s_ref)
        ss_ref[...] = jnp.zeros_like(ss_ref)

    @pl.when(p == 0)
    def _accum():
        s_ref[0:1, :] += jnp.sum(h, axis=0, keepdims=True)
        ss_ref[0:1, :] += jnp.sum(h * h, axis=0, keepdims=True)

    @pl.when(p == 1)
    def _apply():
        mean = s_ref[0:1, :] / N
        var = ss_ref[0:1, :] / N - mean * mean
        t = (h - mean) * (g_ref[...] / jnp.sqrt(var + 1e-5)) + be_ref[...]
        t = jnp.where(t > 0, t, jnp.exp(t) - 1.0)
        o_ref[...] = t + x_ref[...]


def _bn_elu_res(parts_lo, parts_hi, x, bias2d, gamma2d, beta2d):
    return pl.pallas_call(
        _bn_body,
        grid=(2, G),
        in_specs=[
            pl.BlockSpec((1, RB, DH), lambda p, i: (0, i, 0)),
            pl.BlockSpec((1, RB, DH), lambda p, i: (1, i, 0)),
            pl.BlockSpec((1, RB, DH), lambda p, i: (0, i, 0)),
            pl.BlockSpec((1, RB, DH), lambda p, i: (1, i, 0)),
            pl.BlockSpec((RB, D), lambda p, i: (i, 0)),
            pl.BlockSpec((1, D), lambda p, i: (0, 0)),
            pl.BlockSpec((1, D), lambda p, i: (0, 0)),
            pl.BlockSpec((1, D), lambda p, i: (0, 0)),
        ],
        out_specs=pl.BlockSpec((RB, D), lambda p, i: (i, 0)),
        out_shape=jax.ShapeDtypeStruct((N, D), jnp.float32),
        scratch_shapes=[
            pltpu.VMEM((8, D), jnp.float32),
            pltpu.VMEM((8, D), jnp.float32),
        ],
    )(parts_lo, parts_lo, parts_hi, parts_hi, x, bias2d, gamma2d, beta2d)


def kernel(x, hyperedge_index, hyperedge_attr, W, att, bias, gamma, beta):
    row = hyperedge_index[0].astype(jnp.int32)
    col = hyperedge_index[1].astype(jnp.int32)
    att2d = att.reshape(2, D)

    xl, ea, s1b, s2b = _project(x, hyperedge_attr, W, att2d)
    s1 = s1b.reshape(N)
    s2 = s2b.reshape(E)

    ex, dpart, bpart, epart = _sc_scores(row, col, s1, s2)
    dinv3, be3, einv3 = _norm_tables(dpart.reshape(NW, N), bpart.reshape(NW, E),
                                     epart.reshape(NW, E))
    dinv = dinv3.reshape(N)
    be = be3.reshape(E)
    einv = einv3.reshape(E)

    w1, w2 = _sc_weights(row, col, ex, be, einv, dinv)

    pad = jnp.full((NW, NCH, KP - K), DUMMY, jnp.int32)
    col3 = jnp.concatenate([col.reshape(NW, NCH, K), pad], axis=2).reshape(-1)
    row3 = jnp.concatenate([row.reshape(NW, NCH, K), pad], axis=2).reshape(-1)
    zeros = jnp.zeros((EP, DH), jnp.float32)

    outn_halves = []
    for h in range(D // DH):
        xl_h = lax.slice(xl, (0, h * DH), (N, (h + 1) * DH))
        oute2 = _sc_spmv(row, col3, w1, xl_h, zeros)
        oute_h = _combine(oute2.reshape(NC, EP, DH))
        outn_halves.append(_sc_spmv(col, row3, w2, oute_h, zeros))

    out = _bn_elu_res(outn_halves[0].reshape(NC, EP, DH),
                      outn_halves[1].reshape(NC, EP, DH), x,
                      bias.reshape(1, D), gamma.reshape(1, D),
                      beta.reshape(1, D))
    return out
